# Initial kernel scaffold; baseline (speedup 1.0000x reference)
import jax
import jax.numpy as jnp
from jax.experimental import pallas as pl


def kernel(x, F0_rows, F0_cols, F0_vals, F1_rows, F1_cols, F1_vals, F2_rows, F2_cols, F2_vals, W, b):
    raise NotImplementedError("write your pallas kernel here")


# speedup vs baseline: 2.6274x; 2.6274x over previous
import functools

import jax
import jax.numpy as jnp
from jax import lax
from jax.experimental import pallas as pl
from jax.experimental.pallas import tpu as pltpu
from jax.experimental.pallas import tpu_sc as plsc

N = 10000
D = 256
DH = D // 2
NC = 2
NS = 16
L = 16
C = 128
E3 = 480000
EPT = 30080
EP = EPT * NS
NCHUNK = EPT // C
RSTRIDE = 624
RSPAN = 640


def _sc_spmm_body(rows_hbm, cols_hbm, vals_hbm, x2_hbm, z_hbm,
                  idx_r, idx_c, vals_v, gath, acc, sem):
    cid = lax.axis_index("c")
    sid = lax.axis_index("s")

    def zero_row(e, _):
        for j in range(DH // L):
            gath[e, pl.ds(j * L, L)] = jnp.zeros((L,), jnp.float32)
        return 0

    lax.fori_loop(0, C, zero_row, 0)
    row0 = sid * RSTRIDE
    for k in range(RSPAN // C):
        pltpu.sync_copy(gath.at[pl.ds(0, C)], acc.at[pl.ds(row0 + k * C, C)])
    plsc.subcore_barrier()

    tile_base = sid * EPT
    col_off = cid * N

    def chunk_body(i, _):
        base = tile_base + i * C
        pltpu.sync_copy(cols_hbm.at[pl.ds(base, C)], idx_c)
        pltpu.sync_copy(rows_hbm.at[pl.ds(base, C)], idx_r)
        pltpu.sync_copy(vals_hbm.at[pl.ds(base, C)], vals_v)
        for g in range(C // L):
            idx_c[pl.ds(g * L, L)] = idx_c[pl.ds(g * L, L)] + col_off
        pltpu.async_copy(x2_hbm.at[idx_c], gath, sem).wait()

        def group_body(g, _):
            vv = vals_v[pl.ds(g * L, L)]
            for lane in range(L):
                e = g * L + lane
                vsp = vv[lane]
                for j in range(DH // L):
                    gath[e, pl.ds(j * L, L)] = gath[e, pl.ds(j * L, L)] * vsp
            return 0

        lax.fori_loop(0, C // L, group_body, 0)
        pltpu.sync_copy(gath, acc.at[idx_r], add=True)
        return 0

    lax.fori_loop(0, NCHUNK, chunk_body, 0)
    plsc.subcore_barrier()

    pltpu.sync_copy(acc.at[pl.ds(row0, RSPAN)],
                    z_hbm.at[cid, pl.ds(row0, RSPAN)])


@jax.jit
def _sc_spmm(rows, cols, vals, x2):
    mesh = plsc.VectorSubcoreMesh(core_axis_name="c", subcore_axis_name="s",
                                  num_cores=NC, num_subcores=NS)
    return pl.kernel(
        _sc_spmm_body,
        out_type=jax.ShapeDtypeStruct((NC, N, DH), jnp.float32),
        mesh=mesh,
        scratch_types=[
            pltpu.VMEM((C,), jnp.int32),
            pltpu.VMEM((C,), jnp.int32),
            pltpu.VMEM((C,), jnp.float32),
            pltpu.VMEM((C, DH), jnp.float32),
            pltpu.VMEM_SHARED((N, DH), jnp.float32),
            pltpu.SemaphoreType.DMA,
        ],
    )(rows, cols, vals, x2)


def _tc_matmul_body(z0_ref, z1_ref, w_ref, b_ref, out_ref):
    a = z0_ref[0]
    c = z1_ref[0]
    w = w_ref[...]
    out_ref[...] = (
        jnp.dot(a, w[:DH], preferred_element_type=jnp.float32)
        + jnp.dot(c, w[DH:], preferred_element_type=jnp.float32)
        + b_ref[...]
    )


@jax.jit
def _tc_matmul(z, w, b2):
    bm = 1000
    grid = (N // bm,)
    return pl.pallas_call(
        _tc_matmul_body,
        grid=grid,
        in_specs=[
            pl.BlockSpec((1, bm, DH), lambda i: (0, i, 0)),
            pl.BlockSpec((1, bm, DH), lambda i: (1, i, 0)),
            pl.BlockSpec((D, D), lambda i: (0, 0)),
            pl.BlockSpec((1, D), lambda i: (0, 0)),
        ],
        out_specs=pl.BlockSpec((bm, D), lambda i: (i, 0)),
        out_shape=jax.ShapeDtypeStruct((N, D), jnp.float32),
    )(z, z, w, b2)


def kernel(x, F0_rows, F0_cols, F0_vals, F1_rows, F1_cols, F1_vals,
           F2_rows, F2_cols, F2_vals, W, b):
    rows = jnp.concatenate([F0_rows, F1_rows, F2_rows])
    cols = jnp.concatenate([F0_cols, F1_cols, F2_cols])
    vals = jnp.concatenate([F0_vals, F1_vals, F2_vals])
    pad = EP - E3
    rows = jnp.concatenate([rows, jnp.zeros((pad,), jnp.int32)])
    cols = jnp.concatenate([cols, jnp.zeros((pad,), jnp.int32)])
    vals = jnp.concatenate([vals, jnp.zeros((pad,), jnp.float32)])
    x2 = jnp.concatenate([x[:, :DH], x[:, DH:]], axis=0)
    z = _sc_spmm(rows, cols, vals, x2)
    return _tc_matmul(z, W, b[None, :])


# speedup vs baseline: 3.2689x; 1.2441x over previous
import jax
import jax.numpy as jnp
from jax import lax
from jax.experimental import pallas as pl
from jax.experimental.pallas import tpu as pltpu
from jax.experimental.pallas import tpu_sc as plsc

N = 10000
D = 256
DH = D // 2
NC = 2
NS = 16
L = 16
C = 128
E3 = 480000
NCHUNK = 236
EPT = NCHUNK * C
EP = EPT * NS
NPAIR = NCHUNK // 2
RSTRIDE = 624
RSPAN = 640


def _sc_spmm_body(rows_hbm, cols_hbm, vals_hbm, x2_hbm, z_hbm,
                  idx_c0, idx_c1, idx_r, vals_v, gath0, gath1, acc,
                  gsem0, gsem1, ssem):
    cid = lax.axis_index("c")
    sid = lax.axis_index("s")
    idx_c = [idx_c0, idx_c1]
    gath = [gath0, gath1]
    gsem = [gsem0, gsem1]

    def zero_row(e, _):
        for j in range(DH // L):
            gath0[e, pl.ds(j * L, L)] = jnp.zeros((L,), jnp.float32)
        return 0

    lax.fori_loop(0, C, zero_row, 0)
    row0 = sid * RSTRIDE
    for k in range(RSPAN // C):
        pltpu.sync_copy(gath0.at[pl.ds(0, C)], acc.at[pl.ds(row0 + k * C, C)])
    plsc.subcore_barrier()

    tile_base = sid * EPT
    col_off = cid * N

    def load_cols(chunk, p):
        pltpu.sync_copy(cols_hbm.at[pl.ds(tile_base + chunk * C, C)],
                        idx_c[p])
        for g in range(C // L):
            idx_c[p][pl.ds(g * L, L)] = idx_c[p][pl.ds(g * L, L)] + col_off

    def half(chunk, p, wait_scatter, prefetch):
        q = 1 - p
        if wait_scatter:
            pltpu.make_async_copy(gath[q], acc.at[idx_r], ssem).wait()
        if prefetch:
            load_cols(chunk + 1, q)
            pltpu.async_copy(x2_hbm.at[idx_c[q]], gath[q], gsem[q])
        base = tile_base + chunk * C
        pltpu.sync_copy(vals_hbm.at[pl.ds(base, C)], vals_v)
        pltpu.sync_copy(rows_hbm.at[pl.ds(base, C)], idx_r)
        pltpu.make_async_copy(x2_hbm.at[idx_c[p]], gath[p], gsem[p]).wait()

        def group_body(g, _):
            vv = vals_v[pl.ds(g * L, L)]
            for lane in range(L):
                e = g * L + lane
                vsp = vv[lane]
                for j in range(DH // L):
                    gath[p][e, pl.ds(j * L, L)] = (
                        gath[p][e, pl.ds(j * L, L)] * vsp)
            return 0

        lax.fori_loop(0, C // L, group_body, 0)
        pltpu.async_copy(gath[p], acc.at[idx_r], ssem, add=True)

    load_cols(0, 0)
    pltpu.async_copy(x2_hbm.at[idx_c[0]], gath[0], gsem[0])
    half(0, 0, wait_scatter=False, prefetch=True)
    half(1, 1, wait_scatter=True, prefetch=True)

    def pair_body(k, _):
        half(2 * k, 0, wait_scatter=True, prefetch=True)
        half(2 * k + 1, 1, wait_scatter=True, prefetch=True)
        return 0

    lax.fori_loop(1, NPAIR - 1, pair_body, 0)
    half(2 * (NPAIR - 1), 0, wait_scatter=True, prefetch=True)
    half(2 * NPAIR - 1, 1, wait_scatter=True, prefetch=False)
    pltpu.make_async_copy(gath[1], acc.at[idx_r], ssem).wait()
    plsc.subcore_barrier()

    pltpu.sync_copy(acc.at[pl.ds(row0, RSPAN)],
                    z_hbm.at[cid, pl.ds(row0, RSPAN)])


@jax.jit
def _sc_spmm(rows, cols, vals, x2):
    mesh = plsc.VectorSubcoreMesh(core_axis_name="c", subcore_axis_name="s",
                                  num_cores=NC, num_subcores=NS)
    return pl.kernel(
        _sc_spmm_body,
        out_type=jax.ShapeDtypeStruct((NC, N, DH), jnp.float32),
        mesh=mesh,
        scratch_types=[
            pltpu.VMEM((C,), jnp.int32),
            pltpu.VMEM((C,), jnp.int32),
            pltpu.VMEM((C,), jnp.int32),
            pltpu.VMEM((C,), jnp.float32),
            pltpu.VMEM((C, DH), jnp.float32),
            pltpu.VMEM((C, DH), jnp.float32),
            pltpu.VMEM_SHARED((N, DH), jnp.float32),
            pltpu.SemaphoreType.DMA,
            pltpu.SemaphoreType.DMA,
            pltpu.SemaphoreType.DMA,
        ],
    )(rows, cols, vals, x2)


def _tc_matmul_body(z0_ref, z1_ref, w_ref, b_ref, out_ref):
    a = z0_ref[0]
    c = z1_ref[0]
    w = w_ref[...]
    out_ref[...] = (
        jnp.dot(a, w[:DH], preferred_element_type=jnp.float32)
        + jnp.dot(c, w[DH:], preferred_element_type=jnp.float32)
        + b_ref[...]
    )


@jax.jit
def _tc_matmul(z, w, b2):
    bm = 1000
    grid = (N // bm,)
    return pl.pallas_call(
        _tc_matmul_body,
        grid=grid,
        in_specs=[
            pl.BlockSpec((1, bm, DH), lambda i: (0, i, 0)),
            pl.BlockSpec((1, bm, DH), lambda i: (1, i, 0)),
            pl.BlockSpec((D, D), lambda i: (0, 0)),
            pl.BlockSpec((1, D), lambda i: (0, 0)),
        ],
        out_specs=pl.BlockSpec((bm, D), lambda i: (i, 0)),
        out_shape=jax.ShapeDtypeStruct((N, D), jnp.float32),
    )(z, z, w, b2)


def kernel(x, F0_rows, F0_cols, F0_vals, F1_rows, F1_cols, F1_vals,
           F2_rows, F2_cols, F2_vals, W, b):
    rows = jnp.concatenate([F0_rows, F1_rows, F2_rows])
    cols = jnp.concatenate([F0_cols, F1_cols, F2_cols])
    vals = jnp.concatenate([F0_vals, F1_vals, F2_vals])
    pad = EP - E3
    rows = jnp.concatenate([rows, jnp.zeros((pad,), jnp.int32)])
    cols = jnp.concatenate([cols, jnp.zeros((pad,), jnp.int32)])
    vals = jnp.concatenate([vals, jnp.zeros((pad,), jnp.float32)])
    x2 = jnp.concatenate([x[:, :DH], x[:, DH:]], axis=0)
    z = _sc_spmm(rows, cols, vals, x2)
    return _tc_matmul(z, W, b[None, :])
